# baseline (speedup 1.0000x reference)
import functools

import jax
import jax.numpy as jnp
from jax import lax
from jax.experimental import pallas as pl
from jax.experimental.pallas import tpu as pltpu
from jax.experimental.pallas import tpu_sc as plsc

VOCAB = 1000000
DIM = 64
B = 4096
L = 200

NC = 2
NS = 16
NW = NC * NS
BLK = B // NW
TOK_W = BLK * L
NBUF = 2
NGRP = BLK // 16


@functools.partial(
    pl.kernel,
    out_type=jax.ShapeDtypeStruct((L, DIM // 8, NW, 8, BLK), jnp.float32),
    mesh=plsc.VectorSubcoreMesh(core_axis_name="c", subcore_axis_name="s"),
    compiler_params=pltpu.CompilerParams(use_tc_tiling_on_sc=True,
                                         needs_layout_passes=False),
    scratch_types=[
        pltpu.VMEM((TOK_W,), jnp.int32),
        pltpu.VMEM((NBUF, BLK, DIM), jnp.float32),
        pltpu.VMEM((NBUF, DIM // 8, 1, 8, BLK), jnp.float32),
        pltpu.SemaphoreType.DMA,
        pltpu.SemaphoreType.DMA,
        pltpu.SemaphoreType.DMA,
        pltpu.SemaphoreType.DMA,
    ],
)
def _emb_lookup(idx_hbm, table_hbm, out_hbm, idx_v, gbuf, tbuf,
                g0, g1, w0, w1):
    gsem = (g0, g1)
    wsem = (w0, w1)
    wid = lax.axis_index("s") * NC + lax.axis_index("c")
    b0 = wid * BLK

    pltpu.sync_copy(idx_hbm.at[pl.ds(b0 * L, TOK_W)], idx_v)

    lane = lax.iota(jnp.int32, 16)
    lane_l = lane * L
    lane_d = lane * DIM
    zero = lane * 0

    def work(u, bu, do_transpose):
        def group(j0, _):
            v = plsc.load_gather(idx_v, [lane_l + (j0 * (16 * L) + u)])
            for k in range(16):
                pltpu.make_async_copy(
                    table_hbm.at[v[k]],
                    gbuf.at[bu, j0 * 16 + k],
                    gsem[bu]).start()
            if do_transpose:
                def col(d, _):
                    w = plsc.load_gather(
                        gbuf, [zero + (1 - bu), lane + j0 * 16, zero + d])
                    tbuf[1 - bu, d // 8, 0, d % 8, pl.ds(j0 * 16, 16)] = w
                    return 0
                lax.fori_loop(0, DIM, col, 0)
            return 0
        lax.fori_loop(0, NGRP, group, 0)

    def drain_unit(bu):
        pltpu.make_async_copy(
            table_hbm.at[pl.ds(0, BLK)], gbuf.at[bu], gsem[bu]).wait()

    def transpose_unit(bu):
        def grp(j0, _):
            def col(d, _):
                w = plsc.load_gather(
                    gbuf, [zero + bu, lane + j0 * 16, zero + d])
                tbuf[bu, d // 8, 0, d % 8, pl.ds(j0 * 16, 16)] = w
                return 0
            lax.fori_loop(0, DIM, col, 0)
            return 0
        lax.fori_loop(0, NGRP, grp, 0)

    def write_copy(u, bu):
        return pltpu.make_async_copy(
            tbuf.at[bu],
            out_hbm.at[u, pl.ds(0, DIM // 8), pl.ds(wid, 1)],
            wsem[bu])

    work(0, 0, False)

    def step(u2, _):
        for bb in range(NBUF):
            u = u2 * NBUF + bb + 1

            @pl.when(u >= 3)
            def _():
                write_copy(u - 3, bb).wait()

            drain_unit(bb)
            work(u, 1 - bb, True)
            write_copy(u - 1, bb).start()
        return 0

    lax.fori_loop(0, (L - 2) // NBUF, step, 0)

    write_copy(196, 0).wait()
    drain_unit(0)
    work(199, 1, True)
    write_copy(198, 0).start()
    write_copy(197, 1).wait()
    drain_unit(1)
    transpose_unit(1)
    write_copy(199, 1).start()
    write_copy(198, 0).wait()
    write_copy(199, 1).wait()


def kernel(token_ids, table):
    out5 = _emb_lookup(token_ids.reshape(-1), table)
    return out5.transpose(2, 4, 0, 1, 3).reshape(B, L, DIM)


# speedup vs baseline: 1.7770x; 1.7770x over previous
import functools

import jax
import jax.numpy as jnp
from jax import lax
from jax.experimental import pallas as pl
from jax.experimental.pallas import tpu as pltpu
from jax.experimental.pallas import tpu_sc as plsc

VOCAB = 1000000
DIM = 64
B = 4096
L = 200

NC = 2
NS = 16
NW = NC * NS
ROWS_W = B // NW
TOK_W = ROWS_W * L
NBUF = 2
NG = L // 16
REM = L % 16


@functools.partial(
    pl.kernel,
    out_type=jax.ShapeDtypeStruct((B, L, DIM), jnp.float32),
    mesh=plsc.VectorSubcoreMesh(core_axis_name="c", subcore_axis_name="s"),
    compiler_params=pltpu.CompilerParams(use_tc_tiling_on_sc=True),
    scratch_types=[
        pltpu.VMEM((TOK_W + 16,), jnp.int32),
        pltpu.VMEM((NBUF, L, DIM), jnp.float32),
        pltpu.SemaphoreType.DMA,
        pltpu.SemaphoreType.DMA,
        pltpu.SemaphoreType.DMA,
        pltpu.SemaphoreType.DMA,
    ],
)
def _emb_lookup(idx_hbm, table_hbm, out_hbm, idx_v, rows_v, g0, g1, w0, w1):
    gsem = (g0, g1)
    wsem = (w0, w1)
    wid = lax.axis_index("s") * NC + lax.axis_index("c")
    row0 = wid * ROWS_W
    base = row0 * L

    pltpu.sync_copy(idx_hbm.at[pl.ds(base, TOK_W)], idx_v.at[pl.ds(0, TOK_W)])

    def enqueue_row(i, b):
        def group(g, _):
            pos = i * L + g * 16
            v = idx_v[pl.ds(pos, 16)]
            for j in range(16):
                pltpu.make_async_copy(
                    table_hbm.at[v[j]],
                    rows_v.at[b, g * 16 + j],
                    gsem[b]).start()
            return 0
        lax.fori_loop(0, NG, group, 0)
        v = idx_v[pl.ds(i * L + NG * 16, 16)]
        for j in range(REM):
            pltpu.make_async_copy(
                table_hbm.at[v[j]],
                rows_v.at[b, NG * 16 + j],
                gsem[b]).start()

    def drain_row(b):
        pltpu.make_async_copy(
            table_hbm.at[pl.ds(0, L)], rows_v.at[b], gsem[b]).wait()

    def write_copy(i, b):
        return pltpu.make_async_copy(
            rows_v.at[b], out_hbm.at[row0 + i], wsem[b])

    def step(i2, _):
        for b in range(NBUF):
            i = i2 * NBUF + b

            @pl.when(i >= 2)
            def _():
                write_copy(i - 2, b).wait()

            enqueue_row(i, b)

            @pl.when(i >= 1)
            def _():
                drain_row(1 - b)
                write_copy(i - 1, 1 - b).start()
        return 0

    lax.fori_loop(0, ROWS_W // NBUF, step, 0)

    drain_row(1)
    write_copy(ROWS_W - 1, 1).start()
    write_copy(ROWS_W - 2, 0).wait()
    write_copy(ROWS_W - 1, 1).wait()


def kernel(token_ids, table):
    return _emb_lookup(token_ids.reshape(-1), table)
